# baseline (speedup 1.0000x reference)
import functools

import jax
import jax.numpy as jnp
from jax import lax
from jax.experimental import pallas as pl
from jax.experimental.pallas import tpu as pltpu
from jax.experimental.pallas import tpu_sc as plsc

N = 10000
E = 320000
D = 128
NCLS = 40
NC = 2
NS = 16
NW = NC * NS
EB = E // 128
NR = 80
EBP = NW * NR
EPAD = (EBP - EB) * 128
HPAD = 64
TB = EB - (NW - 1) * NR
RPW = 624
TAIL = N - NS * RPW


def _mesh():
    return plsc.VectorSubcoreMesh(core_axis_name="c", subcore_axis_name="s")


def _load_my_blocks(g3_hbm, tails_hbm, plane, buf, wid):
    @pl.when(wid < NW - 1)
    def _():
        pltpu.sync_copy(g3_hbm.at[plane, pl.ds(wid * NR, NR)], buf)

    @pl.when(wid == NW - 1)
    def _():
        pltpu.sync_copy(tails_hbm.at[plane, pl.ds(0, NR)], buf)


def _sc_degree(g3, tails):
    @functools.partial(
        pl.kernel,
        mesh=_mesh(),
        out_type=jax.ShapeDtypeStruct((NC * N, 128), jnp.float32),
        scratch_types=[
            pltpu.VMEM((NR, 128), jnp.int32),
            pltpu.VMEM((128, 128), jnp.float32),
            pltpu.VMEM_SHARED((N, 128), jnp.float32),
            pltpu.SemaphoreType.DMA,
        ],
    )
    def k(g3_hbm, tails_hbm, out_hbm, dst_all, ones_v, deg_sh, sem):
        c = lax.axis_index("c")
        s = lax.axis_index("s")
        wid = c * NS + s
        zeros16 = jnp.zeros((16,), jnp.float32)
        ones16 = jnp.ones((16,), jnp.float32)

        def fill0(r, carry):
            for j in range(8):
                ones_v[r, pl.ds(j * 16, 16)] = zeros16
            return carry

        lax.fori_loop(0, 128, fill0, 0)
        base = s * RPW
        for kk in range(4):
            pltpu.sync_copy(ones_v, deg_sh.at[pl.ds(base + kk * 128, 128)])
        pltpu.sync_copy(ones_v.at[pl.ds(0, RPW - 512)],
                        deg_sh.at[pl.ds(base + 512, RPW - 512)])

        @pl.when(s == 0)
        def _():
            pltpu.sync_copy(ones_v.at[pl.ds(0, TAIL)],
                            deg_sh.at[pl.ds(NS * RPW, TAIL)])

        def fill1(r, carry):
            for j in range(8):
                ones_v[r, pl.ds(j * 16, 16)] = ones16
            return carry

        lax.fori_loop(0, 128, fill1, 0)
        _load_my_blocks(g3_hbm, tails_hbm, 1, dst_all, wid)
        plsc.subcore_barrier()

        def fire(i, carry):
            pltpu.async_copy(ones_v, deg_sh.at[dst_all.at[i]], sem, add=True)
            return carry

        lax.fori_loop(0, NR, fire, 0)

        def drain(i, carry):
            pltpu.make_async_copy(ones_v, deg_sh.at[dst_all.at[0]],
                                  sem).wait()
            return carry

        lax.fori_loop(0, NR, drain, 0)
        plsc.subcore_barrier()
        pltpu.sync_copy(deg_sh.at[pl.ds(base, RPW)],
                        out_hbm.at[pl.ds(c * N + base, RPW)])

        @pl.when(s == 0)
        def _():
            pltpu.sync_copy(deg_sh.at[pl.ds(NS * RPW, TAIL)],
                            out_hbm.at[pl.ds(c * N + NS * RPW, TAIL)])

    return k(g3, tails)


def _sc_gather_scatter(hn, g3, tails):
    NB = 3

    @functools.partial(
        pl.kernel,
        mesh=_mesh(),
        out_type=jax.ShapeDtypeStruct((NC * N, D), jnp.float32),
        scratch_types=(
            [pltpu.VMEM((128,), jnp.int32)] * NB
            + [pltpu.VMEM((128,), jnp.int32)] * NB
            + [pltpu.VMEM((128, D), jnp.float32)] * NB
            + [pltpu.VMEM_SHARED((N, D), jnp.float32)]
            + [pltpu.SemaphoreType.DMA] * (4 * NB)
        ),
    )
    def k(hn_hbm, g3_hbm, tails_hbm, out_hbm, *refs):
        sbuf = refs[0:NB]
        dbuf = refs[NB:2 * NB]
        rows = refs[2 * NB:3 * NB]
        agg_sh = refs[3 * NB]
        gsem = refs[3 * NB + 1:3 * NB + 1 + NB]
        ssem = refs[3 * NB + 1 + NB:3 * NB + 1 + 2 * NB]
        isem = refs[3 * NB + 1 + 2 * NB:3 * NB + 1 + 3 * NB]
        dsem = refs[3 * NB + 1 + 3 * NB:3 * NB + 1 + 4 * NB]
        c = lax.axis_index("c")
        s = lax.axis_index("s")
        wid = c * NS + s
        r0 = wid * NR
        zeros16 = jnp.zeros((16,), jnp.float32)

        def load_idx_row(plane, idx, buf, sem):
            @pl.when(wid < NW - 1)
            def _():
                pltpu.async_copy(g3_hbm.at[plane, r0 + idx], buf, sem)

            @pl.when(wid == NW - 1)
            def _():
                pltpu.async_copy(tails_hbm.at[plane, idx], buf, sem)

        def wait_idx(buf, sem):
            pltpu.make_async_copy(g3_hbm.at[0, r0], buf, sem).wait()

        def zb(r, carry):
            for j in range(8):
                rows[0][r, pl.ds(j * 16, 16)] = zeros16
            return carry

        lax.fori_loop(0, 128, zb, 0)
        base = s * RPW
        for kk in range(4):
            pltpu.sync_copy(rows[0], agg_sh.at[pl.ds(base + kk * 128, 128)])
        pltpu.sync_copy(rows[0].at[pl.ds(0, RPW - 512)],
                        agg_sh.at[pl.ds(base + 512, RPW - 512)])

        @pl.when(s == 0)
        def _():
            pltpu.sync_copy(rows[0].at[pl.ds(0, TAIL)],
                            agg_sh.at[pl.ds(NS * RPW, TAIL)])

        for b in range(NB):
            load_idx_row(0, b, sbuf[b], isem[b])
            load_idx_row(1, b, dbuf[b], dsem[b])
        for b in range(NB):
            wait_idx(sbuf[b], isem[b])
            pltpu.async_copy(hn_hbm.at[sbuf[b]], rows[b], gsem[b])
        plsc.subcore_barrier()

        def body(t, carry):
            for b in range(NB):
                ch = NB * t + b

                @pl.when(ch < NR)
                def _():
                    pltpu.make_async_copy(hn_hbm.at[sbuf[b]], rows[b],
                                          gsem[b]).wait()

                @pl.when(ch + NB < NR)
                def _():
                    load_idx_row(0, ch + NB, sbuf[b], isem[b])

                @pl.when(ch < NR)
                def _():
                    wait_idx(dbuf[b], dsem[b])
                    pltpu.async_copy(rows[b], agg_sh.at[dbuf[b]], ssem[b],
                                     add=True)

                @pl.when(ch + NB < NR)
                def _():
                    pltpu.make_async_copy(rows[b], agg_sh.at[dbuf[b]],
                                          ssem[b]).wait()
                    load_idx_row(1, ch + NB, dbuf[b], dsem[b])
                    wait_idx(sbuf[b], isem[b])
                    pltpu.async_copy(hn_hbm.at[sbuf[b]], rows[b], gsem[b])

            return carry

        lax.fori_loop(0, -(-NR // NB), body, 0)
        for b in range(NB):
            pltpu.make_async_copy(rows[b], agg_sh.at[dbuf[b]],
                                  ssem[b]).wait()
        plsc.subcore_barrier()
        pltpu.sync_copy(agg_sh.at[pl.ds(base, RPW)],
                        out_hbm.at[pl.ds(c * N + base, RPW)])

        @pl.when(s == 0)
        def _():
            pltpu.sync_copy(agg_sh.at[pl.ds(NS * RPW, TAIL)],
                            out_hbm.at[pl.ds(c * N + NS * RPW, TAIL)])

    return k(hn, g3, tails)


_R = 1000


def _tc_matmul(features, W1):
    def body(f_ref, w_ref, out_ref):
        out_ref[...] = jnp.dot(f_ref[...], w_ref[...],
                               preferred_element_type=jnp.float32)

    return pl.pallas_call(
        body,
        grid=(N // _R,),
        in_specs=[
            pl.BlockSpec((_R, D), lambda i: (i, 0)),
            pl.BlockSpec((D, D), lambda i: (0, 0)),
        ],
        out_specs=pl.BlockSpec((_R, D), lambda i: (i, 0)),
        out_shape=jax.ShapeDtypeStruct((N, D), jnp.float32),
    )(features, W1)


def _tc_norm_scale(degp, u):
    def body(degp_ref, u_ref, norm_ref, hn_ref):
        i = pl.program_id(0)
        d = degp_ref[0, :, 0:1] + degp_ref[1, :, 0:1]
        iota = (lax.broadcasted_iota(jnp.int32, (_R, 1), 0)
                + jnp.minimum(i, 9) * _R)
        d = jnp.where(iota < EPAD, d - 1.0, d)
        nrm2 = lax.rsqrt(jnp.maximum(d, 1.0))
        norm_ref[...] = nrm2
        hn_ref[...] = jnp.where(i < N // _R, u_ref[...] * nrm2, 0.0)

    return pl.pallas_call(
        body,
        grid=(N // _R + 1,),
        in_specs=[
            pl.BlockSpec((NC, _R, 128),
                         lambda i: (0, jnp.minimum(i, N // _R - 1), 0)),
            pl.BlockSpec((_R, D),
                         lambda i: (jnp.minimum(i, N // _R - 1), 0)),
        ],
        out_specs=[
            pl.BlockSpec((_R, 1), lambda i: (jnp.minimum(i, N // _R - 1), 0)),
            pl.BlockSpec((_R, D), lambda i: (i, 0)),
        ],
        out_shape=[
            jax.ShapeDtypeStruct((N, 1), jnp.float32),
            jax.ShapeDtypeStruct((N + _R, D), jnp.float32),
        ],
    )(degp, u)


def _tc_mid(p, norm, b, W):
    def body(p_ref, n_ref, b_ref, w_ref, out_ref):
        i = pl.program_id(0)
        agg = p_ref[0] + p_ref[1]
        h = jnp.maximum(agg * n_ref[...] + b_ref[...][None, :], 0.0)
        hn = jnp.dot(h, w_ref[...],
                     preferred_element_type=jnp.float32) * n_ref[...]
        out_ref[...] = jnp.where(i < N // _R, hn, 0.0)

    return pl.pallas_call(
        body,
        grid=(N // _R + 1,),
        in_specs=[
            pl.BlockSpec((NC, _R, D),
                         lambda i: (0, jnp.minimum(i, N // _R - 1), 0)),
            pl.BlockSpec((_R, 1), lambda i: (jnp.minimum(i, N // _R - 1), 0)),
            pl.BlockSpec((D,), lambda i: (0,)),
            pl.BlockSpec((D, D), lambda i: (0, 0)),
        ],
        out_specs=pl.BlockSpec((_R, D), lambda i: (i, 0)),
        out_shape=jax.ShapeDtypeStruct((N + _R, D), jnp.float32),
    )(p, norm, b, W)


def _tc_final(qflat, norm, b, W3, b3):
    def body(q0_ref, q1_ref, n_ref, b_ref, w3_ref, b3_ref, out_ref):
        agg = q0_ref[...] + q1_ref[...]
        h = jnp.maximum(agg * n_ref[...] + b_ref[...][None, :], 0.0)
        out_ref[...] = jnp.dot(h, w3_ref[...],
                               preferred_element_type=jnp.float32) + b3_ref[...][None, :]

    return pl.pallas_call(
        body,
        grid=(N // _R,),
        in_specs=[
            pl.BlockSpec((_R, D), lambda i: (i, 0)),
            pl.BlockSpec((_R, D), lambda i: (i + N // _R, 0)),
            pl.BlockSpec((_R, 1), lambda i: (i, 0)),
            pl.BlockSpec((D,), lambda i: (0,)),
            pl.BlockSpec((D, NCLS), lambda i: (0, 0)),
            pl.BlockSpec((NCLS,), lambda i: (0,)),
        ],
        out_specs=pl.BlockSpec((_R, NCLS), lambda i: (i, 0)),
        out_shape=jax.ShapeDtypeStruct((N, NCLS), jnp.float32),
    )(qflat, qflat, norm, b, W3, b3)


def kernel(g, features, W1, b1, W2, b2, W3, b3):
    g3 = g.reshape(2, EB, 128)
    pad_iota = jnp.arange(EPAD, dtype=jnp.int32)
    pads = jnp.stack([N + (pad_iota % HPAD), pad_iota]).reshape(
        2, EPAD // 128, 128)
    tails = jnp.concatenate([g3[:, EB - TB:], pads], axis=1)
    degp = _sc_degree(g3, tails).reshape(NC, N, 128)
    u = _tc_matmul(features, W1)
    norm, hn1 = _tc_norm_scale(degp, u)
    p = _sc_gather_scatter(hn1, g3, tails)
    hn2 = _tc_mid(p.reshape(NC, N, D), norm, b1, W2)
    q = _sc_gather_scatter(hn2, g3, tails)
    return _tc_final(q, norm, b2, W3, b3)


# speedup vs baseline: 1.0183x; 1.0183x over previous
import functools

import jax
import jax.numpy as jnp
from jax import lax
from jax.experimental import pallas as pl
from jax.experimental.pallas import tpu as pltpu
from jax.experimental.pallas import tpu_sc as plsc

N = 10000
E = 320000
D = 128
NCLS = 40
NC = 2
NS = 16
NW = NC * NS
EB = E // 128
NR = 80
EBP = NW * NR
EPAD = (EBP - EB) * 128
HPAD = 64
TB = EB - (NW - 1) * NR
RPW = 624
TAIL = N - NS * RPW


def _mesh():
    return plsc.VectorSubcoreMesh(core_axis_name="c", subcore_axis_name="s")


def _load_my_blocks(g3_hbm, tails_hbm, plane, buf, wid):
    @pl.when(wid < NW - 1)
    def _():
        pltpu.sync_copy(g3_hbm.at[plane, pl.ds(wid * NR, NR)], buf)

    @pl.when(wid == NW - 1)
    def _():
        pltpu.sync_copy(tails_hbm.at[plane, pl.ds(0, NR)], buf)


def _sc_degree(g3, tails):
    @functools.partial(
        pl.kernel,
        mesh=_mesh(),
        out_type=jax.ShapeDtypeStruct((NC * N, 128), jnp.float32),
        scratch_types=[
            pltpu.VMEM((NR, 128), jnp.int32),
            pltpu.VMEM((128, 128), jnp.float32),
            pltpu.VMEM_SHARED((N, 128), jnp.float32),
            pltpu.SemaphoreType.DMA,
        ],
    )
    def k(g3_hbm, tails_hbm, out_hbm, dst_all, ones_v, deg_sh, sem):
        c = lax.axis_index("c")
        s = lax.axis_index("s")
        wid = c * NS + s
        zeros16 = jnp.zeros((16,), jnp.float32)
        ones16 = jnp.ones((16,), jnp.float32)

        def fill0(r, carry):
            for j in range(8):
                ones_v[r, pl.ds(j * 16, 16)] = zeros16
            return carry

        lax.fori_loop(0, 128, fill0, 0)
        base = s * RPW
        for kk in range(4):
            pltpu.sync_copy(ones_v, deg_sh.at[pl.ds(base + kk * 128, 128)])
        pltpu.sync_copy(ones_v.at[pl.ds(0, RPW - 512)],
                        deg_sh.at[pl.ds(base + 512, RPW - 512)])

        @pl.when(s == 0)
        def _():
            pltpu.sync_copy(ones_v.at[pl.ds(0, TAIL)],
                            deg_sh.at[pl.ds(NS * RPW, TAIL)])

        def fill1(r, carry):
            for j in range(8):
                ones_v[r, pl.ds(j * 16, 16)] = ones16
            return carry

        lax.fori_loop(0, 128, fill1, 0)
        _load_my_blocks(g3_hbm, tails_hbm, 1, dst_all, wid)
        plsc.subcore_barrier()

        def fire(i, carry):
            pltpu.async_copy(ones_v, deg_sh.at[dst_all.at[i]], sem, add=True)
            return carry

        lax.fori_loop(0, NR, fire, 0)

        def drain(i, carry):
            pltpu.make_async_copy(ones_v, deg_sh.at[dst_all.at[0]],
                                  sem).wait()
            return carry

        lax.fori_loop(0, NR, drain, 0)
        plsc.subcore_barrier()
        pltpu.sync_copy(deg_sh.at[pl.ds(base, RPW)],
                        out_hbm.at[pl.ds(c * N + base, RPW)])

        @pl.when(s == 0)
        def _():
            pltpu.sync_copy(deg_sh.at[pl.ds(NS * RPW, TAIL)],
                            out_hbm.at[pl.ds(c * N + NS * RPW, TAIL)])

    return k(g3, tails)


def _sc_gather_scatter(hn, g3, tails):
    NB = 3

    @functools.partial(
        pl.kernel,
        mesh=_mesh(),
        out_type=jax.ShapeDtypeStruct((NC * N, D), jnp.float32),
        scratch_types=(
            [pltpu.VMEM((128,), jnp.int32)] * NB
            + [pltpu.VMEM((128,), jnp.int32)] * NB
            + [pltpu.VMEM((128, D), jnp.float32)] * NB
            + [pltpu.VMEM_SHARED((N, D), jnp.float32)]
            + [pltpu.SemaphoreType.DMA] * (4 * NB)
        ),
    )
    def k(hn_hbm, g3_hbm, tails_hbm, out_hbm, *refs):
        sbuf = refs[0:NB]
        dbuf = refs[NB:2 * NB]
        rows = refs[2 * NB:3 * NB]
        agg_sh = refs[3 * NB]
        gsem = refs[3 * NB + 1:3 * NB + 1 + NB]
        ssem = refs[3 * NB + 1 + NB:3 * NB + 1 + 2 * NB]
        isem = refs[3 * NB + 1 + 2 * NB:3 * NB + 1 + 3 * NB]
        dsem = refs[3 * NB + 1 + 3 * NB:3 * NB + 1 + 4 * NB]
        c = lax.axis_index("c")
        s = lax.axis_index("s")
        wid = c * NS + s
        r0 = wid * NR
        zeros16 = jnp.zeros((16,), jnp.float32)

        def load_idx_row(plane, idx, buf, sem):
            @pl.when(wid < NW - 1)
            def _():
                pltpu.async_copy(g3_hbm.at[plane, r0 + idx], buf, sem)

            @pl.when(wid == NW - 1)
            def _():
                pltpu.async_copy(tails_hbm.at[plane, idx], buf, sem)

        def wait_idx(buf, sem):
            pltpu.make_async_copy(g3_hbm.at[0, r0], buf, sem).wait()

        def zb(r, carry):
            for j in range(8):
                rows[0][r, pl.ds(j * 16, 16)] = zeros16
            return carry

        lax.fori_loop(0, 128, zb, 0)
        base = s * RPW
        for kk in range(4):
            pltpu.sync_copy(rows[0], agg_sh.at[pl.ds(base + kk * 128, 128)])
        pltpu.sync_copy(rows[0].at[pl.ds(0, RPW - 512)],
                        agg_sh.at[pl.ds(base + 512, RPW - 512)])

        @pl.when(s == 0)
        def _():
            pltpu.sync_copy(rows[0].at[pl.ds(0, TAIL)],
                            agg_sh.at[pl.ds(NS * RPW, TAIL)])

        for b in range(NB):
            load_idx_row(0, b, sbuf[b], isem[b])
            load_idx_row(1, b, dbuf[b], dsem[b])
        for b in range(NB):
            wait_idx(sbuf[b], isem[b])
            pltpu.async_copy(hn_hbm.at[sbuf[b]], rows[b], gsem[b])
        plsc.subcore_barrier()

        def body(t, carry):
            for b in range(NB):
                ch = NB * t + b

                @pl.when(ch < NR)
                def _():
                    pltpu.make_async_copy(hn_hbm.at[sbuf[b]], rows[b],
                                          gsem[b]).wait()

                @pl.when(ch + NB < NR)
                def _():
                    load_idx_row(0, ch + NB, sbuf[b], isem[b])

                @pl.when(ch < NR)
                def _():
                    wait_idx(dbuf[b], dsem[b])
                    pltpu.async_copy(rows[b], agg_sh.at[dbuf[b]], ssem[b],
                                     add=True)

                @pl.when(ch + NB < NR)
                def _():
                    pltpu.make_async_copy(rows[b], agg_sh.at[dbuf[b]],
                                          ssem[b]).wait()
                    load_idx_row(1, ch + NB, dbuf[b], dsem[b])
                    wait_idx(sbuf[b], isem[b])
                    pltpu.async_copy(hn_hbm.at[sbuf[b]], rows[b], gsem[b])

            return carry

        lax.fori_loop(0, -(-NR // NB), body, 0)
        for b in range(NB):
            pltpu.make_async_copy(rows[b], agg_sh.at[dbuf[b]],
                                  ssem[b]).wait()
        plsc.subcore_barrier()
        pltpu.sync_copy(agg_sh.at[pl.ds(base, RPW)],
                        out_hbm.at[pl.ds(c * N + base, RPW)])

        @pl.when(s == 0)
        def _():
            pltpu.sync_copy(agg_sh.at[pl.ds(NS * RPW, TAIL)],
                            out_hbm.at[pl.ds(c * N + NS * RPW, TAIL)])

    return k(hn, g3, tails)


_R = 1000


def _tc_matmul(features, W1):
    def body(f_ref, w_ref, out_ref):
        out_ref[...] = jnp.dot(f_ref[...], w_ref[...],
                               preferred_element_type=jnp.float32)

    return pl.pallas_call(
        body,
        grid=(N // _R,),
        in_specs=[
            pl.BlockSpec((_R, D), lambda i: (i, 0)),
            pl.BlockSpec((D, D), lambda i: (0, 0)),
        ],
        out_specs=pl.BlockSpec((_R, D), lambda i: (i, 0)),
        out_shape=jax.ShapeDtypeStruct((N, D), jnp.float32),
    )(features, W1)


def _tc_norm_scale(degp, u):
    def body(degp_ref, u_ref, norm_ref, hn_ref):
        d = degp_ref[0, :, 0:1] + degp_ref[1, :, 0:1]
        iota = lax.broadcasted_iota(jnp.int32, (N, 1), 0)
        d = jnp.where(iota < EPAD, d - 1.0, d)
        nrm2 = lax.rsqrt(jnp.maximum(d, 1.0))
        norm_ref[...] = nrm2
        hn_ref[...] = jnp.concatenate(
            [u_ref[...] * nrm2, jnp.zeros((HPAD, D), jnp.float32)], axis=0)

    return pl.pallas_call(
        body,
        grid=(1,),
        in_specs=[
            pl.BlockSpec((NC, N, 128), lambda i: (0, 0, 0)),
            pl.BlockSpec((N, D), lambda i: (0, 0)),
        ],
        out_specs=[
            pl.BlockSpec((N, 1), lambda i: (0, 0)),
            pl.BlockSpec((N + HPAD, D), lambda i: (0, 0)),
        ],
        out_shape=[
            jax.ShapeDtypeStruct((N, 1), jnp.float32),
            jax.ShapeDtypeStruct((N + HPAD, D), jnp.float32),
        ],
    )(degp, u)


def _tc_mid(p, norm, b, W):
    def body(p_ref, n_ref, b_ref, w_ref, out_ref):
        agg = p_ref[0] + p_ref[1]
        h = jnp.maximum(agg * n_ref[...] + b_ref[...][None, :], 0.0)
        hn = jnp.dot(h, w_ref[...],
                     preferred_element_type=jnp.float32) * n_ref[...]
        out_ref[...] = jnp.concatenate(
            [hn, jnp.zeros((HPAD, D), jnp.float32)], axis=0)

    return pl.pallas_call(
        body,
        grid=(1,),
        in_specs=[
            pl.BlockSpec((NC, N, D), lambda i: (0, 0, 0)),
            pl.BlockSpec((N, 1), lambda i: (0, 0)),
            pl.BlockSpec((D,), lambda i: (0,)),
            pl.BlockSpec((D, D), lambda i: (0, 0)),
        ],
        out_specs=pl.BlockSpec((N + HPAD, D), lambda i: (0, 0)),
        out_shape=jax.ShapeDtypeStruct((N + HPAD, D), jnp.float32),
    )(p, norm, b, W)


def _tc_final(qflat, norm, b, W3, b3):
    def body(q0_ref, q1_ref, n_ref, b_ref, w3_ref, b3_ref, out_ref):
        agg = q0_ref[...] + q1_ref[...]
        h = jnp.maximum(agg * n_ref[...] + b_ref[...][None, :], 0.0)
        out_ref[...] = jnp.dot(h, w3_ref[...],
                               preferred_element_type=jnp.float32) + b3_ref[...][None, :]

    return pl.pallas_call(
        body,
        grid=(N // _R,),
        in_specs=[
            pl.BlockSpec((_R, D), lambda i: (i, 0)),
            pl.BlockSpec((_R, D), lambda i: (i + N // _R, 0)),
            pl.BlockSpec((_R, 1), lambda i: (i, 0)),
            pl.BlockSpec((D,), lambda i: (0,)),
            pl.BlockSpec((D, NCLS), lambda i: (0, 0)),
            pl.BlockSpec((NCLS,), lambda i: (0,)),
        ],
        out_specs=pl.BlockSpec((_R, NCLS), lambda i: (i, 0)),
        out_shape=jax.ShapeDtypeStruct((N, NCLS), jnp.float32),
    )(qflat, qflat, norm, b, W3, b3)


def kernel(g, features, W1, b1, W2, b2, W3, b3):
    g3 = g.reshape(2, EB, 128)
    pad_iota = jnp.arange(EPAD, dtype=jnp.int32)
    pads = jnp.stack([N + (pad_iota % HPAD), pad_iota]).reshape(
        2, EPAD // 128, 128)
    tails = jnp.concatenate([g3[:, EB - TB:], pads], axis=1)
    degp = _sc_degree(g3, tails).reshape(NC, N, 128)
    u = _tc_matmul(features, W1)
    norm, hn1 = _tc_norm_scale(degp, u)
    p = _sc_gather_scatter(hn1, g3, tails)
    hn2 = _tc_mid(p.reshape(NC, N, D), norm, b1, W2)
    q = _sc_gather_scatter(hn2, g3, tails)
    return _tc_final(q, norm, b2, W3, b3)
